# baseline (speedup 1.0000x reference)
import functools

import jax
import jax.numpy as jnp
import numpy as np
from jax import lax
from jax.experimental import pallas as pl
from jax.experimental.pallas import tpu as pltpu
from jax.experimental.pallas import tpu_sc as plsc

_SPATIAL = ((64, 64), (32, 32), (16, 16), (8, 8))
_OFFSETS = (0, 4096, 5120, 5376)

_NC, _NS = 2, 16
_NW = _NC * _NS

_BQ = 64


def _prep_body(lx_ref, ly_ref, aw_ref, idx_ref, wts_ref):
    nh = 8
    b = pl.program_id(0)
    lane = lax.broadcasted_iota(jnp.int32, (_BQ, 128), 1)
    lvl = (lane // 4) % 4
    h = lane // 16
    wi = 64 >> lvl
    off = jnp.where(lvl == 0, 0,
                    jnp.where(lvl == 1, _OFFSETS[1],
                              jnp.where(lvl == 2, _OFFSETS[2], _OFFSETS[3])))
    wf = wi.astype(jnp.float32)

    lx = lx_ref[...]
    ly = ly_ref[...]
    aw = aw_ref[...]

    gx = lx * wf - 0.5
    gy = ly * wf - 0.5
    x0f = jnp.floor(gx)
    y0f = jnp.floor(gy)
    fx = gx - x0f
    fy = gy - y0f
    x0 = x0f.astype(jnp.int32)
    y0 = y0f.astype(jnp.int32)
    wm1 = wi - 1
    xc0 = jnp.maximum(x0, 0)
    xc1 = jnp.minimum(x0 + 1, wm1)
    yc0 = jnp.maximum(y0, 0)
    yc1 = jnp.minimum(y0 + 1, wm1)
    zero = jnp.zeros_like(fx)
    wx0 = jnp.where(x0 >= 0, 1.0 - fx, zero)
    wx1 = jnp.where(x0 + 1 <= wm1, fx, zero)
    wy0 = jnp.where(y0 >= 0, 1.0 - fy, zero)
    wy1 = jnp.where(y0 + 1 <= wm1, fy, zero)

    base = b * (5440 * nh) + off * nh + h
    wn = wi * nh
    row_y0 = base + yc0 * wn
    row_y1 = base + yc1 * wn
    idx_ref[0] = row_y0 + xc0 * nh
    idx_ref[1] = row_y0 + xc1 * nh
    idx_ref[2] = row_y1 + xc0 * nh
    idx_ref[3] = row_y1 + xc1 * nh
    wts_ref[0] = wx0 * wy0 * aw
    wts_ref[1] = wx1 * wy0 * aw
    wts_ref[2] = wx0 * wy1 * aw
    wts_ref[3] = wx1 * wy1 * aw


def _prep(locx, locy, aw, bs, nq):
    rows = bs * nq
    nblk = nq // _BQ
    grid = (bs, nblk)
    in_spec = pl.BlockSpec((_BQ, 128), lambda b, j, nblk=nblk: (b * nblk + j, 0))
    out_spec = pl.BlockSpec((4, _BQ, 128), lambda b, j, nblk=nblk: (0, b * nblk + j, 0))
    return pl.pallas_call(
        _prep_body,
        grid=grid,
        in_specs=[in_spec, in_spec, in_spec],
        out_specs=[out_spec, out_spec],
        out_shape=[
            jax.ShapeDtypeStruct((4, rows, 128), jnp.int32),
            jax.ShapeDtypeStruct((4, rows, 128), jnp.float32),
        ],
    )(locx, locy, aw)


def _make_sc_gather(n_rows, rw, blk):
    ng = rw // blk
    kpb = blk * 64
    assert kpb <= 128
    mesh = plsc.VectorSubcoreMesh(
        core_axis_name="c", subcore_axis_name="s",
        num_cores=_NC, num_subcores=_NS)

    @functools.partial(
        pl.kernel,
        out_type=jax.ShapeDtypeStruct((n_rows, 32), jnp.float32),
        mesh=mesh,
        compiler_params=pltpu.CompilerParams(use_tc_tiling_on_sc=False),
        scratch_types=[
            pltpu.VMEM((2, kpb), jnp.int32),
            pltpu.VMEM((2, kpb), jnp.float32),
            pltpu.VMEM((2, kpb, 32), jnp.float32),
            pltpu.VMEM((2, blk, 32), jnp.float32),
            pltpu.SemaphoreType.DMA((2,)),
            pltpu.SemaphoreType.DMA((2,)),
            pltpu.SemaphoreType.DMA((2,)),
            pltpu.SemaphoreType.DMA((2,)),
        ],
    )
    def sc_kern(value_hbm, idx_hbm, wts_hbm, out_hbm,
                idxb, wtsb, rows, outb, idx_sem, wts_sem, gat_sem, out_sem):
        wid = lax.axis_index("s") * _NC + lax.axis_index("c")
        wbase = wid * rw

        def idx_copy(slot, g):
            e0 = (wbase + g * blk) * 64
            return pltpu.make_async_copy(
                idx_hbm.at[pl.ds(e0, kpb)], idxb.at[slot], idx_sem.at[slot])

        def wts_copy(slot, g):
            e0 = (wbase + g * blk) * 64
            return pltpu.make_async_copy(
                wts_hbm.at[pl.ds(e0, kpb)], wtsb.at[slot], wts_sem.at[slot])

        def gat_copy(slot):
            return pltpu.make_async_copy(
                value_hbm.at[idxb.at[slot]], rows.at[slot], gat_sem.at[slot])

        def out_copy(slot, g):
            r0 = wbase + g * blk
            return pltpu.make_async_copy(
                outb.at[slot], out_hbm.at[pl.ds(r0, blk), :], out_sem.at[slot])

        def compute(slot):
            for r in range(blk):
                acc0 = jnp.zeros((16,), jnp.float32)
                acc1 = jnp.zeros((16,), jnp.float32)
                for kc in range(4):
                    wch = wtsb[slot, pl.ds(r * 64 + kc * 16, 16)]
                    for j in range(16):
                        q = r * 64 + kc * 16 + j
                        w = wch[j]
                        acc0 = acc0 + rows[slot, q, 0:16] * w
                        acc1 = acc1 + rows[slot, q, 16:32] * w
                outb[slot, r, 0:16] = acc0
                outb[slot, r, 16:32] = acc1

        def process(g, cur):
            nxt = 1 - cur

            @pl.when(g + 1 < ng)
            def _():
                idx_copy(nxt, g + 1).wait()
                wts_copy(nxt, g + 1).wait()
                gat_copy(nxt).start()

            gat_copy(cur).wait()

            @pl.when(g + 2 < ng)
            def _():
                idx_copy(cur, g + 2).start()
                wts_copy(cur, g + 2).start()

            @pl.when(g >= 2)
            def _():
                out_copy(cur, g - 2).wait()

            compute(cur)
            out_copy(cur, g).start()

        idx_copy(0, 0).start()
        wts_copy(0, 0).start()
        idx_copy(0, 0).wait()
        wts_copy(0, 0).wait()
        gat_copy(0).start()
        idx_copy(1, 1).start()
        wts_copy(1, 1).start()

        def loop_body(i, carry):
            process(i * 2, 0)
            process(i * 2 + 1, 1)
            return carry

        lax.fori_loop(0, ng // 2, loop_body, 0)

        out_copy(0, ng - 2).wait()
        out_copy(1, ng - 1).wait()

    return sc_kern


def kernel(value, value_spatial_shapes, sampling_locations, attention_weights):
    bs, nk, nh, dh = value.shape
    nq = sampling_locations.shape[1]
    n_rows = bs * nq * nh

    locx = sampling_locations[..., 0].reshape(bs * nq, 128)
    locy = sampling_locations[..., 1].reshape(bs * nq, 128)
    aw = attention_weights.reshape(bs * nq, 128)

    idx4, wts4 = _prep(locx, locy, aw, bs, nq)
    idx = idx4.reshape(4, bs * nq, nh, 16).transpose(1, 2, 0, 3).reshape(n_rows * 64)
    wts = wts4.reshape(4, bs * nq, nh, 16).transpose(1, 2, 0, 3).reshape(n_rows * 64)

    value_flat = value.reshape(bs * nk * nh, dh)
    rw = n_rows // _NW
    sc = _make_sc_gather(n_rows, rw, 2)
    out = sc(value_flat, idx, wts)
    return out.reshape(bs, nq, nh * dh)


# speedup vs baseline: 1.0131x; 1.0131x over previous
import functools

import jax
import jax.numpy as jnp
import numpy as np
from jax import lax
from jax.experimental import pallas as pl
from jax.experimental.pallas import tpu as pltpu
from jax.experimental.pallas import tpu_sc as plsc

_SPATIAL = ((64, 64), (32, 32), (16, 16), (8, 8))
_OFFSETS = (0, 4096, 5120, 5376)

_NC, _NS = 2, 16
_NW = _NC * _NS

_BQ = 64


def _prep_body(lx_ref, ly_ref, aw_ref, idx_ref, wts_ref):
    nh = 8
    b = pl.program_id(0)
    lane = lax.broadcasted_iota(jnp.int32, (_BQ, 128), 1)
    lvl = (lane // 4) % 4
    h = lane // 16
    wi = 64 >> lvl
    off = jnp.where(lvl == 0, 0,
                    jnp.where(lvl == 1, _OFFSETS[1],
                              jnp.where(lvl == 2, _OFFSETS[2], _OFFSETS[3])))
    wf = wi.astype(jnp.float32)

    lx = lx_ref[...]
    ly = ly_ref[...]
    aw = aw_ref[...]

    gx = lx * wf - 0.5
    gy = ly * wf - 0.5
    x0f = jnp.floor(gx)
    y0f = jnp.floor(gy)
    fx = gx - x0f
    fy = gy - y0f
    x0 = x0f.astype(jnp.int32)
    y0 = y0f.astype(jnp.int32)
    wm1 = wi - 1
    xc0 = jnp.maximum(x0, 0)
    xc1 = jnp.minimum(x0 + 1, wm1)
    yc0 = jnp.maximum(y0, 0)
    yc1 = jnp.minimum(y0 + 1, wm1)
    zero = jnp.zeros_like(fx)
    wx0 = jnp.where(x0 >= 0, 1.0 - fx, zero)
    wx1 = jnp.where(x0 + 1 <= wm1, fx, zero)
    wy0 = jnp.where(y0 >= 0, 1.0 - fy, zero)
    wy1 = jnp.where(y0 + 1 <= wm1, fy, zero)

    base = b * (5440 * nh) + off * nh + h
    wn = wi * nh
    row_y0 = base + yc0 * wn
    row_y1 = base + yc1 * wn
    idx_ref[0] = row_y0 + xc0 * nh
    idx_ref[1] = row_y0 + xc1 * nh
    idx_ref[2] = row_y1 + xc0 * nh
    idx_ref[3] = row_y1 + xc1 * nh

    wts_ref[0] = wx0 * wy0 * aw
    wts_ref[1] = wx1 * wy0 * aw
    wts_ref[2] = wx0 * wy1 * aw
    wts_ref[3] = wx1 * wy1 * aw


def _prep(locx, locy, aw, bs, nq):
    rows = bs * nq
    nblk = nq // _BQ
    grid = (bs, nblk)
    in_spec = pl.BlockSpec((_BQ, 128), lambda b, j, nblk=nblk: (b * nblk + j, 0))
    out_spec = pl.BlockSpec((4, _BQ, 128), lambda b, j, nblk=nblk: (0, b * nblk + j, 0))
    return pl.pallas_call(
        _prep_body,
        grid=grid,
        in_specs=[in_spec, in_spec, in_spec],
        out_specs=[out_spec, out_spec],
        out_shape=[
            jax.ShapeDtypeStruct((4, rows, 128), jnp.int32),
            jax.ShapeDtypeStruct((4, rows, 128), jnp.float32),
        ],
    )(locx, locy, aw)


def _make_sc_gather(n_rows, rw, blk):
    ng = rw // blk
    kpb = blk * 64
    ngat = kpb // 128
    assert kpb % 128 == 0
    mesh = plsc.VectorSubcoreMesh(
        core_axis_name="c", subcore_axis_name="s",
        num_cores=_NC, num_subcores=_NS)

    @functools.partial(
        pl.kernel,
        out_type=jax.ShapeDtypeStruct((n_rows, 32), jnp.float32),
        mesh=mesh,
        compiler_params=pltpu.CompilerParams(use_tc_tiling_on_sc=False),
        scratch_types=[
            pltpu.VMEM((2, kpb), jnp.int32),
            pltpu.VMEM((2, kpb), jnp.float32),
            pltpu.VMEM((2, kpb, 16), jnp.int32),
            pltpu.VMEM((2, blk, 32), jnp.float32),
            pltpu.SemaphoreType.DMA((2,)),
            pltpu.SemaphoreType.DMA((2,)),
            pltpu.SemaphoreType.DMA((2,)),
            pltpu.SemaphoreType.DMA((2,)),
        ],
    )
    def sc_kern(value_hbm, idx_hbm, wts_hbm, out_hbm,
                idxb, wtsb, rows, outb, idx_sem, wts_sem, gat_sem, out_sem):
        wid = lax.axis_index("s") * _NC + lax.axis_index("c")
        wbase = wid * rw

        def idx_copy(slot, g):
            e0 = (wbase + g * blk) * 64
            return pltpu.make_async_copy(
                idx_hbm.at[pl.ds(e0, kpb)], idxb.at[slot], idx_sem.at[slot])

        def wts_copy(slot, g):
            e0 = (wbase + g * blk) * 64
            return pltpu.make_async_copy(
                wts_hbm.at[pl.ds(e0, kpb)], wtsb.at[slot], wts_sem.at[slot])

        def gat_copy(slot, j):
            return pltpu.make_async_copy(
                value_hbm.at[idxb.at[slot, pl.ds(j * 128, 128)]],
                rows.at[slot, pl.ds(j * 128, 128), :], gat_sem.at[slot])

        def out_copy(slot, g):
            r0 = wbase + g * blk
            return pltpu.make_async_copy(
                outb.at[slot], out_hbm.at[pl.ds(r0, blk), :], out_sem.at[slot])

        def compute(slot):
            for r in range(blk):
                acc0 = jnp.zeros((16,), jnp.float32)
                acc1 = jnp.zeros((16,), jnp.float32)
                for kc in range(4):
                    wch = wtsb[slot, pl.ds(r * 64 + kc * 16, 16)]
                    for j in range(16):
                        q = r * 64 + kc * 16 + j
                        w = wch[j]
                        word = rows[slot, q, :]
                        a = lax.bitcast_convert_type(word << 16, jnp.float32)
                        b = lax.bitcast_convert_type(
                            word & jnp.int32(-65536), jnp.float32)
                        acc0 = acc0 + a * w
                        acc1 = acc1 + b * w
                outb[slot, r, 0:16] = acc0
                outb[slot, r, 16:32] = acc1

        def process(g, cur):
            nxt = 1 - cur

            @pl.when(g + 1 < ng)
            def _():
                idx_copy(nxt, g + 1).wait()
                wts_copy(nxt, g + 1).wait()
                for j in range(ngat):
                    gat_copy(nxt, j).start()

            for j in range(ngat):
                gat_copy(cur, j).wait()

            @pl.when(g + 2 < ng)
            def _():
                idx_copy(cur, g + 2).start()
                wts_copy(cur, g + 2).start()

            @pl.when(g >= 2)
            def _():
                out_copy(cur, g - 2).wait()

            compute(cur)
            out_copy(cur, g).start()

        idx_copy(0, 0).start()
        wts_copy(0, 0).start()
        idx_copy(0, 0).wait()
        wts_copy(0, 0).wait()
        for j in range(ngat):
            gat_copy(0, j).start()
        idx_copy(1, 1).start()
        wts_copy(1, 1).start()

        def loop_body(i, carry):
            process(i * 2, 0)
            process(i * 2 + 1, 1)
            return carry

        lax.fori_loop(0, ng // 2, loop_body, 0)

        out_copy(0, ng - 2).wait()
        out_copy(1, ng - 1).wait()

    return sc_kern


def kernel(value, value_spatial_shapes, sampling_locations, attention_weights):
    bs, nk, nh, dh = value.shape
    nq = sampling_locations.shape[1]
    n_rows = bs * nq * nh

    locx = sampling_locations[..., 0].reshape(bs * nq, 128)
    locy = sampling_locations[..., 1].reshape(bs * nq, 128)
    aw = attention_weights.reshape(bs * nq, 128)

    idx4, wts4 = _prep(locx, locy, aw, bs, nq)
    idx = idx4.reshape(4, bs * nq, nh, 16).transpose(1, 2, 0, 3).reshape(n_rows * 64)
    wts = wts4.reshape(4, bs * nq, nh, 16).transpose(1, 2, 0, 3).reshape(n_rows * 64)

    value_flat = value.reshape(bs * nk * nh, 2, dh // 2)
    value_bf = lax.bitcast_convert_type(
        value_flat.astype(jnp.bfloat16).transpose(0, 2, 1), jnp.int32)
    rw = n_rows // _NW
    sc = _make_sc_gather(n_rows, rw, 4)
    out = sc(value_bf, idx, wts)
    return out.reshape(bs, nq, nh * dh)


# speedup vs baseline: 1.6407x; 1.6195x over previous
import functools

import jax
import jax.numpy as jnp
import numpy as np
from jax import lax
from jax.experimental import pallas as pl
from jax.experimental.pallas import tpu as pltpu
from jax.experimental.pallas import tpu_sc as plsc

_SPATIAL = ((64, 64), (32, 32), (16, 16), (8, 8))
_OFFSETS = (0, 4096, 5120, 5376)

_NC, _NS = 2, 16
_NW = _NC * _NS

_BQ = 64


def _prep_body(lx_ref, ly_ref, aw_ref, idx_ref, wts_ref):
    nh = 8
    b = pl.program_id(0)
    lane = lax.broadcasted_iota(jnp.int32, (_BQ, 128), 1)
    lvl = (lane // 4) % 4
    h = lane // 16
    wi = 64 >> lvl
    off = jnp.where(lvl == 0, 0,
                    jnp.where(lvl == 1, _OFFSETS[1],
                              jnp.where(lvl == 2, _OFFSETS[2], _OFFSETS[3])))
    wf = wi.astype(jnp.float32)

    lx = lx_ref[...]
    ly = ly_ref[...]
    aw = aw_ref[...]

    gx = lx * wf - 0.5
    gy = ly * wf - 0.5
    x0f = jnp.floor(gx)
    y0f = jnp.floor(gy)
    fx = gx - x0f
    fy = gy - y0f
    x0 = x0f.astype(jnp.int32)
    y0 = y0f.astype(jnp.int32)
    wm1 = wi - 1
    xc0 = jnp.maximum(x0, 0)
    xc1 = jnp.minimum(x0 + 1, wm1)
    yc0 = jnp.maximum(y0, 0)
    yc1 = jnp.minimum(y0 + 1, wm1)
    zero = jnp.zeros_like(fx)
    wx0 = jnp.where(x0 >= 0, 1.0 - fx, zero)
    wx1 = jnp.where(x0 + 1 <= wm1, fx, zero)
    wy0 = jnp.where(y0 >= 0, 1.0 - fy, zero)
    wy1 = jnp.where(y0 + 1 <= wm1, fy, zero)

    base = (b * nh + h) * 5440 + off
    row_y0 = base + yc0 * wi
    row_y1 = base + yc1 * wi
    idx_ref[0] = row_y0 + xc0
    idx_ref[1] = row_y0 + xc1
    idx_ref[2] = row_y1 + xc0
    idx_ref[3] = row_y1 + xc1

    wts_ref[0] = wx0 * wy0 * aw
    wts_ref[1] = wx1 * wy0 * aw
    wts_ref[2] = wx0 * wy1 * aw
    wts_ref[3] = wx1 * wy1 * aw


def _prep(locx, locy, aw, bs, nq):
    rows = bs * nq
    nblk = nq // _BQ
    grid = (bs, nblk)
    in_spec = pl.BlockSpec((_BQ, 128), lambda b, j, nblk=nblk: (b * nblk + j, 0))
    out_spec = pl.BlockSpec((4, _BQ, 128), lambda b, j, nblk=nblk: (0, b * nblk + j, 0))
    return pl.pallas_call(
        _prep_body,
        grid=grid,
        in_specs=[in_spec, in_spec, in_spec],
        out_specs=[out_spec, out_spec],
        out_shape=[
            jax.ShapeDtypeStruct((4, rows, 128), jnp.int32),
            jax.ShapeDtypeStruct((4, rows, 128), jnp.float32),
        ],
    )(locx, locy, aw)


def _make_sc_gather(n_rows, rw, blk):
    ng = rw // blk
    kpb = blk * 64
    ngat = kpb // 128
    assert kpb % 128 == 0
    mesh = plsc.VectorSubcoreMesh(
        core_axis_name="c", subcore_axis_name="s",
        num_cores=_NC, num_subcores=_NS)

    @functools.partial(
        pl.kernel,
        out_type=jax.ShapeDtypeStruct((n_rows * 32,), jnp.float32),
        mesh=mesh,
        compiler_params=pltpu.CompilerParams(use_tc_tiling_on_sc=False),
        scratch_types=[
            pltpu.VMEM((2, kpb), jnp.int32),
            pltpu.VMEM((2, kpb), jnp.float32),
            pltpu.VMEM((2, kpb, 16), jnp.int32),
            pltpu.VMEM((2, blk * 32), jnp.float32),
            pltpu.SemaphoreType.DMA((2,)),
            pltpu.SemaphoreType.DMA((2,)),
            pltpu.SemaphoreType.DMA((2,)),
            pltpu.SemaphoreType.DMA((2,)),
        ],
    )
    def sc_kern(value_hbm, idx_hbm, wts_hbm, out_hbm,
                idxb, wtsb, rows, outb, idx_sem, wts_sem, gat_sem, out_sem):
        wid = lax.axis_index("s") * _NC + lax.axis_index("c")
        wbase = wid * rw

        def idx_copy(slot, g):
            e0 = (wbase + g * blk) * 64
            return pltpu.make_async_copy(
                idx_hbm.at[pl.ds(e0, kpb)], idxb.at[slot], idx_sem.at[slot])

        def wts_copy(slot, g):
            e0 = (wbase + g * blk) * 64
            return pltpu.make_async_copy(
                wts_hbm.at[pl.ds(e0, kpb)], wtsb.at[slot], wts_sem.at[slot])

        def gat_copy(slot, j):
            return pltpu.make_async_copy(
                value_hbm.at[idxb.at[slot, pl.ds(j * 128, 128)]],
                rows.at[slot, pl.ds(j * 128, 128), :], gat_sem.at[slot])

        def out_copy(slot, g):
            r0 = wbase + g * blk
            return pltpu.make_async_copy(
                outb.at[slot], out_hbm.at[pl.ds(r0 * 32, blk * 32)],
                out_sem.at[slot])

        def compute(slot):
            def row_body(r, carry):
                part0 = []
                part1 = []
                for kc in range(4):
                    wch = wtsb[slot, pl.ds(r * 64 + kc * 16, 16)]
                    e0 = jnp.zeros((16,), jnp.float32)
                    e1 = jnp.zeros((16,), jnp.float32)
                    f0 = jnp.zeros((16,), jnp.float32)
                    f1 = jnp.zeros((16,), jnp.float32)
                    for j in range(16):
                        q = r * 64 + kc * 16 + j
                        w = wch[j]
                        word = rows[slot, q, :]
                        a = lax.bitcast_convert_type(word << 16, jnp.float32)
                        b = lax.bitcast_convert_type(
                            word & jnp.int32(-65536), jnp.float32)
                        if j % 2 == 0:
                            e0 = e0 + a * w
                            e1 = e1 + b * w
                        else:
                            f0 = f0 + a * w
                            f1 = f1 + b * w
                    part0.append(e0 + f0)
                    part1.append(e1 + f1)
                acc0 = (part0[0] + part0[1]) + (part0[2] + part0[3])
                acc1 = (part1[0] + part1[1]) + (part1[2] + part1[3])
                outb[slot, pl.ds(r * 32, 16)] = acc0
                outb[slot, pl.ds(r * 32 + 16, 16)] = acc1
                return carry

            lax.fori_loop(0, blk, row_body, 0)

        def process(g, cur):
            nxt = 1 - cur

            @pl.when(g + 1 < ng)
            def _():
                idx_copy(nxt, g + 1).wait()
                wts_copy(nxt, g + 1).wait()
                for j in range(ngat):
                    gat_copy(nxt, j).start()

            for j in range(ngat):
                gat_copy(cur, j).wait()

            @pl.when(g + 2 < ng)
            def _():
                idx_copy(cur, g + 2).start()
                wts_copy(cur, g + 2).start()

            @pl.when(g >= 2)
            def _():
                out_copy(cur, g - 2).wait()

            compute(cur)
            out_copy(cur, g).start()

        idx_copy(0, 0).start()
        wts_copy(0, 0).start()
        idx_copy(0, 0).wait()
        wts_copy(0, 0).wait()
        for j in range(ngat):
            gat_copy(0, j).start()
        idx_copy(1, 1).start()
        wts_copy(1, 1).start()

        def loop_body(i, carry):
            process(i * 2, 0)
            process(i * 2 + 1, 1)
            return carry

        lax.fori_loop(0, ng // 2, loop_body, 0)

        out_copy(0, ng - 2).wait()
        out_copy(1, ng - 1).wait()

    return sc_kern


def kernel(value, value_spatial_shapes, sampling_locations, attention_weights):
    bs, nk, nh, dh = value.shape
    nq = sampling_locations.shape[1]
    n_rows = bs * nq * nh

    locx = sampling_locations[..., 0].reshape(bs * nq, 128)
    locy = sampling_locations[..., 1].reshape(bs * nq, 128)
    aw = attention_weights.reshape(bs * nq, 128)

    idx4, wts4 = _prep(locx, locy, aw, bs, nq)
    idx = idx4.reshape(4, bs * nq, nh, 16).transpose(1, 2, 0, 3).reshape(n_rows * 64)
    wts = wts4.reshape(4, bs * nq, nh, 16).transpose(1, 2, 0, 3).reshape(n_rows * 64)

    value_flat = value.transpose(0, 2, 1, 3).reshape(bs * nh * nk, 2, dh // 2)
    value_bf = lax.bitcast_convert_type(
        value_flat.astype(jnp.bfloat16).transpose(0, 2, 1), jnp.int32)
    rw = n_rows // _NW
    sc = _make_sc_gather(n_rows, rw, 16)
    out = sc(value_bf, idx, wts)
    return out.reshape(bs, nq, nh * dh)
